# baseline (speedup 1.0000x reference)
import jax
import jax.numpy as jnp
from jax import lax
from jax.experimental import pallas as pl
from jax.experimental.pallas import tpu as pltpu, tpu_sc as plsc
import functools

_NUM_PRODUCTS = 1000000
_D = 128
_B = 16384

_NC = 2
_NS = 16
_NW = _NC * _NS
_BPW = _B // _NW
_CH = 128
_NCK = _BPW // _CH
_NV = _D // 16

_mesh = plsc.VectorSubcoreMesh(core_axis_name="c", subcore_axis_name="s")


@functools.partial(
    pl.kernel,
    out_type=jax.ShapeDtypeStruct((_B, _D), jnp.float32),
    mesh=_mesh,
    scratch_types=[
        pltpu.VMEM((_BPW,), jnp.int32),
        pltpu.VMEM((_NCK, _CH, _D), jnp.float32),
        pltpu.SemaphoreType.DMA,
        pltpu.SemaphoreType.DMA,
        pltpu.SemaphoreType.DMA,
        pltpu.SemaphoreType.DMA,
        pltpu.SemaphoreType.DMA,
    ],
    compiler_params=pltpu.CompilerParams(needs_layout_passes=False),
)
def _lookup_normalize(idx_hbm, table_hbm, out_hbm, idx_v, rows_v,
                      g0, g1, g2, g3, s_out):
    wid = lax.axis_index("s") * _NC + lax.axis_index("c")
    base = wid * _BPW

    gsems = (g0, g1, g2, g3)
    pltpu.sync_copy(idx_hbm.at[pl.ds(base, _CH)], idx_v.at[pl.ds(0, _CH)])
    gcps = [
        pltpu.async_copy(table_hbm.at[idx_v.at[pl.ds(0, _CH)]],
                         rows_v.at[0], gsems[0])
    ]
    pltpu.sync_copy(idx_hbm.at[pl.ds(base + _CH, _CH * (_NCK - 1))],
                    idx_v.at[pl.ds(_CH, _CH * (_NCK - 1))])
    for j in range(1, _NCK):
        gcps.append(
            pltpu.async_copy(table_hbm.at[idx_v.at[pl.ds(j * _CH, _CH)]],
                             rows_v.at[j], gsems[j]))

    lanes = lax.iota(jnp.int32, 16)
    dnums = lax.GatherDimensionNumbers(
        offset_dims=(), collapsed_slice_dims=(0,), start_index_map=(0,))
    perms = [((lanes + s) & 15)[:, None] for s in (8, 4, 2, 1)]

    out_cps = []
    for j in range(_NCK):
        gcps[j].wait()

        @plsc.parallel_loop(0, _CH, 1, unroll=4)
        def row_body(r, j=j):
            vs = [rows_v[j, r, pl.ds(k * 16, 16)] for k in range(_NV)]
            sq = [v * v for v in vs]
            while len(sq) > 1:
                sq = [sq[i] + sq[i + 1] for i in range(0, len(sq), 2)]
            acc = sq[0]
            for p in perms:
                rot = lax.gather(
                    acc, p, dnums, slice_sizes=(1,),
                    mode=lax.GatherScatterMode.PROMISE_IN_BOUNDS)
                acc = acc + rot
            tv = jnp.maximum(acc, 1e-24)
            yi = jnp.int32(0x5F3759DF) - (plsc.bitcast(tv, jnp.int32) >> 1)
            y = plsc.bitcast(yi, jnp.float32)
            y = y * (1.5 - (tv * 0.5) * y * y)
            for k in range(_NV):
                rows_v[j, r, pl.ds(k * 16, 16)] = vs[k] * y

        out_cps.append(
            pltpu.async_copy(
                rows_v.at[j], out_hbm.at[pl.ds(base + j * _CH, _CH)], s_out
            )
        )

    for cp in out_cps:
        cp.wait()


def kernel(product_ids, table):
    return _lookup_normalize(product_ids.astype(jnp.int32), table)


# speedup vs baseline: 1.0012x; 1.0012x over previous
import jax
import jax.numpy as jnp
from jax import lax
from jax.experimental import pallas as pl
from jax.experimental.pallas import tpu as pltpu, tpu_sc as plsc
import functools

_NUM_PRODUCTS = 1000000
_D = 128
_B = 16384

_NC = 2
_NS = 16
_NW = _NC * _NS
_BPW = _B // _NW
_CH = 128
_NCK = _BPW // _CH
_NV = _D // 16

_mesh = plsc.VectorSubcoreMesh(core_axis_name="c", subcore_axis_name="s")


@functools.partial(
    pl.kernel,
    out_type=jax.ShapeDtypeStruct((_B, _D), jnp.float32),
    mesh=_mesh,
    scratch_types=[
        pltpu.VMEM((_NCK, _CH), jnp.int32),
        pltpu.VMEM((_NCK, _CH, _D), jnp.float32),
        pltpu.SemaphoreType.DMA,
        pltpu.SemaphoreType.DMA,
        pltpu.SemaphoreType.DMA,
        pltpu.SemaphoreType.DMA,
        pltpu.SemaphoreType.DMA,
    ],
    compiler_params=pltpu.CompilerParams(needs_layout_passes=False),
)
def _lookup_normalize(idx_hbm, table_hbm, out_hbm, idx_v, rows_v,
                      g0, g1, g2, g3, s_out):
    wid = lax.axis_index("s") * _NC + lax.axis_index("c")
    base = wid * _BPW

    gsems = (g0, g1, g2, g3)
    pltpu.sync_copy(idx_hbm.at[wid, 0], idx_v.at[0])
    gcps = [
        pltpu.async_copy(table_hbm.at[idx_v.at[0]], rows_v.at[0], gsems[0])
    ]
    pltpu.sync_copy(idx_hbm.at[wid, pl.ds(1, _NCK - 1)],
                    idx_v.at[pl.ds(1, _NCK - 1)])
    for j in range(1, _NCK):
        gcps.append(
            pltpu.async_copy(table_hbm.at[idx_v.at[j]], rows_v.at[j],
                             gsems[j]))

    lanes = lax.iota(jnp.int32, 16)
    dnums = lax.GatherDimensionNumbers(
        offset_dims=(), collapsed_slice_dims=(0,), start_index_map=(0,))
    perms = [((lanes + s) & 15)[:, None] for s in (8, 4, 2, 1)]

    out_cps = []
    for j in range(_NCK):
        gcps[j].wait()

        @plsc.parallel_loop(0, _CH, 1, unroll=4)
        def row_body(r, j=j):
            vs = [rows_v[j, r, pl.ds(k * 16, 16)] for k in range(_NV)]
            sq = [v * v for v in vs]
            while len(sq) > 1:
                sq = [sq[i] + sq[i + 1] for i in range(0, len(sq), 2)]
            acc = sq[0]
            for p in perms:
                rot = lax.gather(
                    acc, p, dnums, slice_sizes=(1,),
                    mode=lax.GatherScatterMode.PROMISE_IN_BOUNDS)
                acc = acc + rot
            tv = jnp.maximum(acc, 1e-24)
            yi = jnp.int32(0x5F3759DF) - (plsc.bitcast(tv, jnp.int32) >> 1)
            y = plsc.bitcast(yi, jnp.float32)
            y = y * (1.5 - (tv * 0.5) * y * y)
            for k in range(_NV):
                rows_v[j, r, pl.ds(k * 16, 16)] = vs[k] * y

        out_cps.append(
            pltpu.async_copy(
                rows_v.at[j], out_hbm.at[pl.ds(base + j * _CH, _CH)], s_out
            )
        )

    for cp in out_cps:
        cp.wait()


def kernel(product_ids, table):
    ids = product_ids.astype(jnp.int32).reshape(_NW, _NCK, _CH)
    return _lookup_normalize(ids, table)
